# baseline (speedup 1.0000x reference)
import functools

import jax
import jax.numpy as jnp
from jax import lax
from jax.experimental import pallas as pl
from jax.experimental.pallas import tpu as pltpu
from jax.experimental.pallas import tpu_sc as plsc

_B = 1024
_N = 8192
_D = 64
_BETA = 0.25
_NB = 1024
_TI = 16


def _argmin_body(x_ref, x2_ref, w_ref, w2_ref, ind_ref, bestv_ref, besti_ref):
    j = pl.program_id(0)
    mm = lax.dot_general(
        x_ref[...],
        w_ref[...],
        (((1,), (1,)), ((), ())),
        preferred_element_type=jnp.float32,
    )
    d = (x2_ref[...] - 2.0 * mm) + w2_ref[...]
    bm = jnp.min(d, axis=1, keepdims=True)
    cols = lax.broadcasted_iota(jnp.int32, d.shape, 1)
    masked = jnp.where(d == bm, cols, jnp.int32(2**30))
    bi = jnp.min(masked, axis=1, keepdims=True) + j * _NB

    @pl.when(j == 0)
    def _():
        bestv_ref[...] = bm
        besti_ref[...] = bi

    @pl.when(j > 0)
    def _():
        pv = bestv_ref[...]
        pi = besti_ref[...]
        take = bm < pv
        bestv_ref[...] = jnp.where(take, bm, pv)
        besti_ref[...] = jnp.where(take, bi, pi)

    @pl.when(j == pl.num_programs(0) - 1)
    def _():
        ind_ref[...] = besti_ref[...]


def _top1_indices(x, x2, W, w2):
    return pl.pallas_call(
        _argmin_body,
        grid=(_N // _NB,),
        in_specs=[
            pl.BlockSpec((_B, _D), lambda j: (0, 0)),
            pl.BlockSpec((_B, 1), lambda j: (0, 0)),
            pl.BlockSpec((_NB, _D), lambda j: (j, 0)),
            pl.BlockSpec((1, _NB), lambda j: (0, j)),
        ],
        out_specs=pl.BlockSpec((_B, 1), lambda j: (0, 0)),
        out_shape=jax.ShapeDtypeStruct((_B, 1), jnp.int32),
        scratch_shapes=[
            pltpu.VMEM((_B, 1), jnp.float32),
            pltpu.VMEM((_B, 1), jnp.int32),
        ],
    )(x, x2, W, w2)


def _make_sc_gather():
    info = plsc.get_sparse_core_info()
    nc, ns = info.num_cores, info.num_subcores
    nw = nc * ns
    b_per_w = _B // nw
    mesh = plsc.VectorSubcoreMesh(core_axis_name="c", subcore_axis_name="s")

    @functools.partial(
        pl.kernel,
        mesh=mesh,
        out_type=jax.ShapeDtypeStruct((_B, _D), jnp.float32),
        compiler_params=pltpu.CompilerParams(use_tc_tiling_on_sc=False),
        scratch_types=[
            pltpu.VMEM((b_per_w,), jnp.int32),
            pltpu.VMEM((b_per_w, _D), jnp.float32),
            pltpu.SemaphoreType.DMA,
        ],
    )
    def gather_k(table_hbm, idx_hbm, out_hbm, idx_v, rows_v, sem):
        wid = lax.axis_index("s") * nc + lax.axis_index("c")
        base = wid * b_per_w
        pltpu.sync_copy(idx_hbm.at[pl.ds(base, b_per_w)], idx_v)
        pltpu.async_copy(table_hbm.at[idx_v], rows_v, sem).wait()
        pltpu.sync_copy(rows_v, out_hbm.at[pl.ds(base, b_per_w)])

    return gather_k


_JW = 64


def _bcast_body(q_ref, x_ref, out_ref, loss_ref, pat_ref, sem):
    pat_ref[:, 0, :] = q_ref[...]
    k = 1
    while k < _JW:
        pat_ref[:, pl.ds(k, k), :] = pat_ref[:, pl.ds(0, k), :]
        k *= 2
    nrep = _B // _JW
    for c in range(nrep):
        pltpu.make_async_copy(
            pat_ref, out_ref.at[:, pl.ds(c * _JW, _JW), :], sem
        ).start()
    xw = x_ref[...]
    qb = q_ref[...]
    xsum = jnp.sum(xw, axis=0, keepdims=True)
    qsum = jnp.sum(qb, axis=0, keepdims=True)
    cross = jnp.sum(xsum * qsum, axis=1, keepdims=True)
    sx2 = jnp.sum(jnp.sum(xw * xw, axis=0, keepdims=True), axis=1, keepdims=True)
    sq2 = jnp.sum(jnp.sum(qb * qb, axis=0, keepdims=True), axis=1, keepdims=True)
    total = float(_B) * (sx2 + sq2) - 2.0 * cross
    loss_ref[...] = total * ((1.0 + _BETA) / float(_B * _B * _D))
    for c in range(nrep):
        pltpu.make_async_copy(
            pat_ref, out_ref.at[:, pl.ds(c * _JW, _JW), :], sem
        ).wait()


def _broadcast_and_loss(q, x):
    return pl.pallas_call(
        _bcast_body,
        in_specs=[
            pl.BlockSpec(memory_space=pltpu.VMEM),
            pl.BlockSpec(memory_space=pltpu.VMEM),
        ],
        out_specs=[
            pl.BlockSpec(memory_space=pl.ANY),
            pl.BlockSpec(memory_space=pltpu.VMEM),
        ],
        out_shape=[
            jax.ShapeDtypeStruct((_B, _B, _D), jnp.float32),
            jax.ShapeDtypeStruct((1, 1), jnp.float32),
        ],
        scratch_shapes=[
            pltpu.VMEM((_B, _JW, _D), jnp.float32),
            pltpu.SemaphoreType.DMA,
        ],
    )(q, x)


def kernel(x, W):
    x2 = jnp.sum(x**2, axis=1, keepdims=True)
    w2 = jnp.sum(W**2, axis=1, keepdims=True).T
    ind = _top1_indices(x, x2, W, w2)
    q = _make_sc_gather()(W, ind.reshape((_B,)))
    quantized, loss = _broadcast_and_loss(q, x)
    return quantized, ind, jnp.reshape(loss, ())


# speedup vs baseline: 1.0428x; 1.0428x over previous
import functools

import jax
import jax.numpy as jnp
from jax import lax
from jax.experimental import pallas as pl
from jax.experimental.pallas import tpu as pltpu
from jax.experimental.pallas import tpu_sc as plsc

_B = 1024
_N = 8192
_D = 64
_BETA = 0.25
_NB = 1024
_TI = 16


def _argmin_body(x_ref, x2_ref, w_ref, w2_ref, ind_ref, bestv_ref, besti_ref):
    j = pl.program_id(0)
    mm = lax.dot_general(
        x_ref[...],
        w_ref[...],
        (((1,), (1,)), ((), ())),
        preferred_element_type=jnp.float32,
    )
    d = (x2_ref[...] - 2.0 * mm) + w2_ref[...]
    bm = jnp.min(d, axis=1, keepdims=True)
    cols = lax.broadcasted_iota(jnp.int32, d.shape, 1)
    masked = jnp.where(d == bm, cols, jnp.int32(2**30))
    bi = jnp.min(masked, axis=1, keepdims=True) + j * _NB

    @pl.when(j == 0)
    def _():
        bestv_ref[...] = bm
        besti_ref[...] = bi

    @pl.when(j > 0)
    def _():
        pv = bestv_ref[...]
        pi = besti_ref[...]
        take = bm < pv
        bestv_ref[...] = jnp.where(take, bm, pv)
        besti_ref[...] = jnp.where(take, bi, pi)

    @pl.when(j == pl.num_programs(0) - 1)
    def _():
        ind_ref[...] = besti_ref[...]


def _top1_indices(x, x2, W, w2):
    return pl.pallas_call(
        _argmin_body,
        grid=(_N // _NB,),
        in_specs=[
            pl.BlockSpec((_B, _D), lambda j: (0, 0)),
            pl.BlockSpec((_B, 1), lambda j: (0, 0)),
            pl.BlockSpec((_NB, _D), lambda j: (j, 0)),
            pl.BlockSpec((1, _NB), lambda j: (0, j)),
        ],
        out_specs=pl.BlockSpec((_B, 1), lambda j: (0, 0)),
        out_shape=jax.ShapeDtypeStruct((_B, 1), jnp.int32),
        scratch_shapes=[
            pltpu.VMEM((_B, 1), jnp.float32),
            pltpu.VMEM((_B, 1), jnp.int32),
        ],
    )(x, x2, W, w2)


def _make_sc_gather():
    info = plsc.get_sparse_core_info()
    nc, ns = info.num_cores, info.num_subcores
    nw = nc * ns
    b_per_w = _B // nw
    mesh = plsc.VectorSubcoreMesh(core_axis_name="c", subcore_axis_name="s")

    @functools.partial(
        pl.kernel,
        mesh=mesh,
        out_type=jax.ShapeDtypeStruct((_B, _D), jnp.float32),
        compiler_params=pltpu.CompilerParams(use_tc_tiling_on_sc=False),
        scratch_types=[
            pltpu.VMEM((b_per_w,), jnp.int32),
            pltpu.VMEM((b_per_w, _D), jnp.float32),
            pltpu.SemaphoreType.DMA,
        ],
    )
    def gather_k(table_hbm, idx_hbm, out_hbm, idx_v, rows_v, sem):
        wid = lax.axis_index("s") * nc + lax.axis_index("c")
        base = wid * b_per_w
        pltpu.sync_copy(idx_hbm.at[pl.ds(base, b_per_w)], idx_v)
        pltpu.async_copy(table_hbm.at[idx_v], rows_v, sem).wait()
        pltpu.sync_copy(rows_v, out_hbm.at[pl.ds(base, b_per_w)])

    return gather_k


_JW = 64


def _bcast_body(q_ref, x_ref, out_ref, loss_ref, pat_ref, sem):
    pat_ref[:, pl.ds(0, _D)] = q_ref[...]
    k = _D
    while k < _JW * _D:
        pat_ref[:, pl.ds(k, k)] = pat_ref[:, pl.ds(0, k)]
        k *= 2
    out2 = out_ref
    nrep = _B // _JW
    for c in range(nrep):
        pltpu.make_async_copy(
            pat_ref, out2.at[:, pl.ds(c * _JW * _D, _JW * _D)], sem
        ).start()
    xw = x_ref[...]
    qb = q_ref[...]
    xsum = jnp.sum(xw, axis=0, keepdims=True)
    qsum = jnp.sum(qb, axis=0, keepdims=True)
    cross = jnp.sum(xsum * qsum, axis=1, keepdims=True)
    sx2 = jnp.sum(jnp.sum(xw * xw, axis=0, keepdims=True), axis=1, keepdims=True)
    sq2 = jnp.sum(jnp.sum(qb * qb, axis=0, keepdims=True), axis=1, keepdims=True)
    total = float(_B) * (sx2 + sq2) - 2.0 * cross
    loss_ref[...] = total * ((1.0 + _BETA) / float(_B * _B * _D))
    for c in range(nrep):
        pltpu.make_async_copy(
            pat_ref, out2.at[:, pl.ds(c * _JW * _D, _JW * _D)], sem
        ).wait()


def _broadcast_and_loss(q, x):
    return pl.pallas_call(
        _bcast_body,
        in_specs=[
            pl.BlockSpec(memory_space=pltpu.VMEM),
            pl.BlockSpec(memory_space=pltpu.VMEM),
        ],
        out_specs=[
            pl.BlockSpec(memory_space=pl.ANY),
            pl.BlockSpec(memory_space=pltpu.VMEM),
        ],
        out_shape=[
            jax.ShapeDtypeStruct((_B, _B * _D), jnp.float32),
            jax.ShapeDtypeStruct((1, 1), jnp.float32),
        ],
        scratch_shapes=[
            pltpu.VMEM((_B, _JW * _D), jnp.float32),
            pltpu.SemaphoreType.DMA,
        ],
    )(q, x)


def kernel(x, W):
    x2 = jnp.sum(x**2, axis=1, keepdims=True)
    w2 = jnp.sum(W**2, axis=1, keepdims=True).T
    ind = _top1_indices(x, x2, W, w2)
    q = _make_sc_gather()(W, ind.reshape((_B,)))
    quantized2d, loss = _broadcast_and_loss(q, x)
    quantized = jnp.reshape(quantized2d, (_B, _B, _D))
    return quantized, ind, jnp.reshape(loss, ())


# speedup vs baseline: 4.3012x; 4.1246x over previous
import functools

import jax
import jax.numpy as jnp
from jax import lax
from jax.experimental import pallas as pl
from jax.experimental.pallas import tpu as pltpu
from jax.experimental.pallas import tpu_sc as plsc

_B = 1024
_N = 8192
_D = 64
_BETA = 0.25
_NB = 1024
_TI = 32


def _argmin_body(x_ref, x2_ref, w_ref, w2_ref, ind_ref, bestv_ref, besti_ref):
    j = pl.program_id(0)
    mm = lax.dot_general(
        x_ref[...],
        w_ref[...],
        (((1,), (1,)), ((), ())),
        preferred_element_type=jnp.float32,
    )
    d = (x2_ref[...] - 2.0 * mm) + w2_ref[...]
    bm = jnp.min(d, axis=1, keepdims=True)
    cols = lax.broadcasted_iota(jnp.int32, d.shape, 1)
    masked = jnp.where(d == bm, cols, jnp.int32(2**30))
    bi = jnp.min(masked, axis=1, keepdims=True) + j * _NB

    @pl.when(j == 0)
    def _():
        bestv_ref[...] = bm
        besti_ref[...] = bi

    @pl.when(j > 0)
    def _():
        pv = bestv_ref[...]
        pi = besti_ref[...]
        take = bm < pv
        bestv_ref[...] = jnp.where(take, bm, pv)
        besti_ref[...] = jnp.where(take, bi, pi)

    @pl.when(j == pl.num_programs(0) - 1)
    def _():
        ind_ref[...] = besti_ref[...]


def _top1_indices(x, x2, W, w2):
    return pl.pallas_call(
        _argmin_body,
        grid=(_N // _NB,),
        in_specs=[
            pl.BlockSpec((_B, _D), lambda j: (0, 0)),
            pl.BlockSpec((_B, 1), lambda j: (0, 0)),
            pl.BlockSpec((_NB, _D), lambda j: (j, 0)),
            pl.BlockSpec((1, _NB), lambda j: (0, j)),
        ],
        out_specs=pl.BlockSpec((_B, 1), lambda j: (0, 0)),
        out_shape=jax.ShapeDtypeStruct((_B, 1), jnp.int32),
        scratch_shapes=[
            pltpu.VMEM((_B, 1), jnp.float32),
            pltpu.VMEM((_B, 1), jnp.int32),
        ],
    )(x, x2, W, w2)


def _make_sc_gather():
    info = plsc.get_sparse_core_info()
    nc, ns = info.num_cores, info.num_subcores
    nw = nc * ns
    b_per_w = _B // nw
    mesh = plsc.VectorSubcoreMesh(core_axis_name="c", subcore_axis_name="s")

    @functools.partial(
        pl.kernel,
        mesh=mesh,
        out_type=jax.ShapeDtypeStruct((_B, _D), jnp.float32),
        compiler_params=pltpu.CompilerParams(use_tc_tiling_on_sc=False),
        scratch_types=[
            pltpu.VMEM((b_per_w,), jnp.int32),
            pltpu.VMEM((b_per_w, _D), jnp.float32),
            pltpu.SemaphoreType.DMA,
        ],
    )
    def gather_k(table_hbm, idx_hbm, out_hbm, idx_v, rows_v, sem):
        wid = lax.axis_index("s") * nc + lax.axis_index("c")
        base = wid * b_per_w
        pltpu.sync_copy(idx_hbm.at[pl.ds(base, b_per_w)], idx_v)
        pltpu.async_copy(table_hbm.at[idx_v], rows_v, sem).wait()
        pltpu.sync_copy(rows_v, out_hbm.at[pl.ds(base, b_per_w)])

    return gather_k


def _bcast_body(q_ref, x_ref, out_ref, loss_ref, acc_ref):
    i = pl.program_id(0)
    qb = q_ref[...]
    out_ref[...] = jnp.broadcast_to(qb[:, :, None], (_TI, _D, _B))

    xw = x_ref[...]
    xsum = jnp.sum(xw, axis=0, keepdims=True)
    qsum = jnp.sum(qb, axis=0, keepdims=True)
    cross = jnp.sum(xsum * qsum, axis=1, keepdims=True)
    sx2 = jnp.sum(jnp.sum(xw * xw, axis=0, keepdims=True), axis=1, keepdims=True)
    sq2 = jnp.sum(jnp.sum(qb * qb, axis=0, keepdims=True), axis=1, keepdims=True)
    partial = float(_TI) * sx2 + float(_B) * sq2 - 2.0 * cross

    @pl.when(i == 0)
    def _():
        acc_ref[...] = partial

    @pl.when(i > 0)
    def _():
        acc_ref[...] = acc_ref[...] + partial

    @pl.when(i == pl.num_programs(0) - 1)
    def _():
        loss_ref[...] = acc_ref[...] * ((1.0 + _BETA) / float(_B * _B * _D))


def _broadcast_and_loss(q, x):
    return pl.pallas_call(
        _bcast_body,
        grid=(_B // _TI,),
        in_specs=[
            pl.BlockSpec((_TI, _D), lambda i: (i, 0)),
            pl.BlockSpec((_B, _D), lambda i: (0, 0)),
        ],
        out_specs=[
            pl.BlockSpec((_TI, _D, _B), lambda i: (i, 0, 0)),
            pl.BlockSpec((1, 1), lambda i: (0, 0)),
        ],
        out_shape=[
            jax.ShapeDtypeStruct((_B, _D, _B), jnp.float32),
            jax.ShapeDtypeStruct((1, 1), jnp.float32),
        ],
        scratch_shapes=[pltpu.VMEM((1, 1), jnp.float32)],
    )(q, x)


def kernel(x, W):
    x2 = jnp.sum(x**2, axis=1, keepdims=True)
    w2 = jnp.sum(W**2, axis=1, keepdims=True).T
    ind = _top1_indices(x, x2, W, w2)
    q = _make_sc_gather()(W, ind.reshape((_B,)))
    quantized_t, loss = _broadcast_and_loss(q, x)
    quantized = jnp.transpose(quantized_t, (0, 2, 1))
    return quantized, ind, jnp.reshape(loss, ())


# speedup vs baseline: 4.3354x; 1.0080x over previous
import functools

import jax
import jax.numpy as jnp
from jax import lax
from jax.experimental import pallas as pl
from jax.experimental.pallas import tpu as pltpu
from jax.experimental.pallas import tpu_sc as plsc

_B = 1024
_N = 8192
_D = 64
_BETA = 0.25
_NB = 1024
_TI = 16


def _argmin_body(x_ref, x2_ref, w_ref, w2_ref, ind_ref, bestv_ref, besti_ref):
    j = pl.program_id(0)
    mm = lax.dot_general(
        x_ref[...],
        w_ref[...],
        (((1,), (1,)), ((), ())),
        preferred_element_type=jnp.float32,
    )
    d = (x2_ref[...] - 2.0 * mm) + w2_ref[...]
    bm = jnp.min(d, axis=1, keepdims=True)
    cols = lax.broadcasted_iota(jnp.int32, d.shape, 1)
    masked = jnp.where(d == bm, cols, jnp.int32(2**30))
    bi = jnp.min(masked, axis=1, keepdims=True) + j * _NB

    @pl.when(j == 0)
    def _():
        bestv_ref[...] = bm
        besti_ref[...] = bi

    @pl.when(j > 0)
    def _():
        pv = bestv_ref[...]
        pi = besti_ref[...]
        take = bm < pv
        bestv_ref[...] = jnp.where(take, bm, pv)
        besti_ref[...] = jnp.where(take, bi, pi)

    @pl.when(j == pl.num_programs(0) - 1)
    def _():
        ind_ref[...] = besti_ref[...]


def _top1_indices(x, x2, W, w2):
    return pl.pallas_call(
        _argmin_body,
        grid=(_N // _NB,),
        in_specs=[
            pl.BlockSpec((_B, _D), lambda j: (0, 0)),
            pl.BlockSpec((_B, 1), lambda j: (0, 0)),
            pl.BlockSpec((_NB, _D), lambda j: (j, 0)),
            pl.BlockSpec((1, _NB), lambda j: (0, j)),
        ],
        out_specs=pl.BlockSpec((_B, 1), lambda j: (0, 0)),
        out_shape=jax.ShapeDtypeStruct((_B, 1), jnp.int32),
        scratch_shapes=[
            pltpu.VMEM((_B, 1), jnp.float32),
            pltpu.VMEM((_B, 1), jnp.int32),
        ],
    )(x, x2, W, w2)


def _make_sc_gather():
    info = plsc.get_sparse_core_info()
    nc, ns = info.num_cores, info.num_subcores
    nw = nc * ns
    b_per_w = _B // nw
    mesh = plsc.VectorSubcoreMesh(core_axis_name="c", subcore_axis_name="s")

    @functools.partial(
        pl.kernel,
        mesh=mesh,
        out_type=jax.ShapeDtypeStruct((_B, _D), jnp.float32),
        compiler_params=pltpu.CompilerParams(use_tc_tiling_on_sc=False),
        scratch_types=[
            pltpu.VMEM((b_per_w,), jnp.int32),
            pltpu.VMEM((b_per_w, _D), jnp.float32),
            pltpu.SemaphoreType.DMA,
        ],
    )
    def gather_k(table_hbm, idx_hbm, out_hbm, idx_v, rows_v, sem):
        wid = lax.axis_index("s") * nc + lax.axis_index("c")
        base = wid * b_per_w
        pltpu.sync_copy(idx_hbm.at[pl.ds(base, b_per_w)], idx_v)
        pltpu.async_copy(table_hbm.at[idx_v], rows_v, sem).wait()
        pltpu.sync_copy(rows_v, out_hbm.at[pl.ds(base, b_per_w)])

    return gather_k


def _bcast_body(q_ref, x_ref, out_ref, loss_ref, acc_ref):
    i = pl.program_id(0)
    qb = q_ref[...]
    out_ref[...] = jnp.broadcast_to(qb[:, :, None], (_TI, _D, _B))

    xw = x_ref[...]
    xsum = jnp.sum(xw, axis=0, keepdims=True)
    qsum = jnp.sum(qb, axis=0, keepdims=True)
    cross = jnp.sum(xsum * qsum, axis=1, keepdims=True)
    sx2 = jnp.sum(jnp.sum(xw * xw, axis=0, keepdims=True), axis=1, keepdims=True)
    sq2 = jnp.sum(jnp.sum(qb * qb, axis=0, keepdims=True), axis=1, keepdims=True)
    partial = float(_TI) * sx2 + float(_B) * sq2 - 2.0 * cross

    @pl.when(i == 0)
    def _():
        acc_ref[...] = partial

    @pl.when(i > 0)
    def _():
        acc_ref[...] = acc_ref[...] + partial

    @pl.when(i == pl.num_programs(0) - 1)
    def _():
        loss_ref[...] = acc_ref[...] * ((1.0 + _BETA) / float(_B * _B * _D))


def _broadcast_and_loss(q, x):
    return pl.pallas_call(
        _bcast_body,
        grid=(_B // _TI,),
        in_specs=[
            pl.BlockSpec((_TI, _D), lambda i: (i, 0)),
            pl.BlockSpec((_B, _D), lambda i: (0, 0)),
        ],
        out_specs=[
            pl.BlockSpec((_TI, _D, _B), lambda i: (i, 0, 0)),
            pl.BlockSpec((1, 1), lambda i: (0, 0)),
        ],
        out_shape=[
            jax.ShapeDtypeStruct((_B, _D, _B), jnp.float32),
            jax.ShapeDtypeStruct((1, 1), jnp.float32),
        ],
        scratch_shapes=[pltpu.VMEM((1, 1), jnp.float32)],
    )(q, x)


def kernel(x, W):
    x2 = jnp.sum(x**2, axis=1, keepdims=True)
    w2 = jnp.sum(W**2, axis=1, keepdims=True).T
    ind = _top1_indices(x, x2, W, w2)
    q = _make_sc_gather()(W, ind.reshape((_B,)))
    quantized_t, loss = _broadcast_and_loss(q, x)
    quantized = jnp.transpose(quantized_t, (0, 2, 1))
    return quantized, ind, jnp.reshape(loss, ())
